# baseline (speedup 1.0000x reference)
import functools

import jax
import jax.numpy as jnp
from jax import lax
from jax.experimental import pallas as pl
from jax.experimental.pallas import tpu as pltpu
from jax.experimental.pallas import tpu_sc as plsc

B = 16384
F_DENSE = 64
DIM_S = 32
DIM_T = 32
F_OUT = F_DENSE + DIM_S + DIM_T
NC = 2
NS = 16
NW = NC * NS
ROWS_PER_W = B // NW
IDX_CHUNK = 128
N_CHUNKS = ROWS_PER_W // IDX_CHUNK
GROUPS = ROWS_PER_W // 16


@functools.partial(
    pl.kernel,
    out_type=jax.ShapeDtypeStruct((B, F_OUT), jnp.float32),
    mesh=plsc.VectorSubcoreMesh(core_axis_name="c", subcore_axis_name="s"),
    compiler_params=pltpu.CompilerParams(use_tc_tiling_on_sc=False),
    scratch_types=[
        pltpu.VMEM((ROWS_PER_W,), jnp.float32),
        pltpu.VMEM((ROWS_PER_W,), jnp.float32),
        pltpu.VMEM((N_CHUNKS, IDX_CHUNK), jnp.int32),
        pltpu.VMEM((N_CHUNKS, IDX_CHUNK), jnp.int32),
        pltpu.VMEM((ROWS_PER_W, DIM_S), jnp.float32),
        pltpu.VMEM((ROWS_PER_W, DIM_T), jnp.float32),
        pltpu.SemaphoreType.DMA,
        pltpu.SemaphoreType.DMA,
    ],
)
def _sc_embed(x_hbm, sid_hbm, tid_hbm, w_s_hbm, w_t_hbm, out_hbm,
              sid_v, tid_v, idx_s_v, idx_t_v, emb_s_v, emb_t_v, sem, dsem):
    wid = lax.axis_index("s") * NC + lax.axis_index("c")
    base = wid * ROWS_PER_W
    rows = pl.ds(base, ROWS_PER_W)

    dense = pltpu.async_copy(x_hbm.at[rows, pl.ds(0, F_DENSE)],
                             out_hbm.at[rows, pl.ds(0, F_DENSE)], dsem)
    ids_s = pltpu.async_copy(sid_hbm.at[rows], sid_v, sem)
    ids_t = pltpu.async_copy(tid_hbm.at[rows], tid_v, sem)
    ids_s.wait()
    ids_t.wait()

    for g in range(GROUPS):
        s_ids = sid_v[pl.ds(g * 16, 16)].astype(jnp.int32)
        t_ids = tid_v[pl.ds(g * 16, 16)].astype(jnp.int32)
        j, off = divmod(g * 16, IDX_CHUNK)
        idx_s_v[j, pl.ds(off, 16)] = s_ids
        idx_t_v[j, pl.ds(off, 16)] = t_ids

    copies = []
    for j in range(N_CHUNKS):
        rows_j = pl.ds(j * IDX_CHUNK, IDX_CHUNK)
        copies.append(pltpu.async_copy(
            w_s_hbm.at[idx_s_v.at[j]], emb_s_v.at[rows_j], sem))
        copies.append(pltpu.async_copy(
            w_t_hbm.at[idx_t_v.at[j]], emb_t_v.at[rows_j], sem))
    for c in copies:
        c.wait()

    pltpu.sync_copy(emb_s_v, out_hbm.at[rows, pl.ds(F_DENSE, DIM_S)])
    pltpu.sync_copy(emb_t_v, out_hbm.at[rows, pl.ds(F_DENSE + DIM_S, DIM_T)])
    dense.wait()


def kernel(x, W_s, W_t):
    return _sc_embed(x, x[:, F_DENSE], x[:, F_DENSE + 1], W_s, W_t)


# speedup vs baseline: 3.0359x; 3.0359x over previous
import functools

import jax
import jax.numpy as jnp
from jax import lax
from jax.experimental import pallas as pl
from jax.experimental.pallas import tpu as pltpu
from jax.experimental.pallas import tpu_sc as plsc

B = 16384
F_DENSE = 64
VOCAB_S = 100
DIM = 32
NC = 2
NS = 16
NW = NC * NS
ROWS_PER_W = B // NW
PAIRS_PER_W = 2 * ROWS_PER_W
IDX_CHUNK = 128
N_CHUNKS = PAIRS_PER_W // IDX_CHUNK
GROUPS = PAIRS_PER_W // 16


@functools.partial(
    pl.kernel,
    out_type=jax.ShapeDtypeStruct((2 * B, DIM), jnp.float32),
    mesh=plsc.VectorSubcoreMesh(core_axis_name="c", subcore_axis_name="s"),
    compiler_params=pltpu.CompilerParams(use_tc_tiling_on_sc=False),
    scratch_types=[
        pltpu.VMEM((PAIRS_PER_W,), jnp.float32),
        pltpu.VMEM((PAIRS_PER_W,), jnp.int32),
        pltpu.VMEM((PAIRS_PER_W, DIM), jnp.float32),
        pltpu.SemaphoreType.DMA,
    ],
)
def _sc_embed(ids_hbm, table_hbm, out_hbm, ids_v, idx_v, emb_v, sem):
    wid = lax.axis_index("s") * NC + lax.axis_index("c")
    base = wid * PAIRS_PER_W

    pltpu.sync_copy(ids_hbm.at[pl.ds(base, PAIRS_PER_W)], ids_v)

    for g in range(GROUPS):
        sl = pl.ds(g * 16, 16)
        idx_v[sl] = ids_v[sl].astype(jnp.int32)

    copies = []
    for j in range(N_CHUNKS):
        sl = pl.ds(j * IDX_CHUNK, IDX_CHUNK)
        copies.append(pltpu.async_copy(
            table_hbm.at[idx_v.at[sl]], emb_v.at[sl], sem))
    for c in copies:
        c.wait()

    pltpu.sync_copy(emb_v, out_hbm.at[pl.ds(base, PAIRS_PER_W)])


def kernel(x, W_s, W_t):
    table = jnp.concatenate((W_s, W_t), axis=0)
    ids = jnp.stack((x[:, F_DENSE], x[:, F_DENSE + 1] + VOCAB_S),
                    axis=1).reshape(2 * B)
    emb = _sc_embed(ids, table)
    return jnp.concatenate((x[:, :F_DENSE], emb.reshape(B, 2 * DIM)), axis=1)


# speedup vs baseline: 3.2325x; 1.0648x over previous
import functools

import jax
import jax.numpy as jnp
from jax import lax
from jax.experimental import pallas as pl
from jax.experimental.pallas import tpu as pltpu
from jax.experimental.pallas import tpu_sc as plsc

B = 16384
F_DENSE = 64
DIM_S = 32
DIM_T = 32
NC = 2
NS = 16
NW = NC * NS
ROWS_PER_W = B // NW
IDX_CHUNK = 512
N_CHUNKS = ROWS_PER_W // IDX_CHUNK
GROUPS = ROWS_PER_W // 16


@functools.partial(
    pl.kernel,
    out_type=(jax.ShapeDtypeStruct((B, DIM_S), jnp.float32),
              jax.ShapeDtypeStruct((B, DIM_T), jnp.float32)),
    mesh=plsc.VectorSubcoreMesh(core_axis_name="c", subcore_axis_name="s"),
    compiler_params=pltpu.CompilerParams(use_tc_tiling_on_sc=False),
    scratch_types=[
        pltpu.VMEM((ROWS_PER_W,), jnp.float32),
        pltpu.VMEM((ROWS_PER_W,), jnp.float32),
        pltpu.VMEM((ROWS_PER_W,), jnp.int32),
        pltpu.VMEM((ROWS_PER_W,), jnp.int32),
        pltpu.VMEM((ROWS_PER_W, DIM_S), jnp.float32),
        pltpu.VMEM((ROWS_PER_W, DIM_T), jnp.float32),
        pltpu.SemaphoreType.DMA,
    ],
)
def _sc_embed(sid_hbm, tid_hbm, w_s_hbm, w_t_hbm, out_s_hbm, out_t_hbm,
              sid_v, tid_v, idx_s_v, idx_t_v, emb_s_v, emb_t_v, sem):
    wid = lax.axis_index("s") * NC + lax.axis_index("c")
    base = wid * ROWS_PER_W
    rows = pl.ds(base, ROWS_PER_W)

    ids_s = pltpu.async_copy(sid_hbm.at[rows], sid_v, sem)
    ids_t = pltpu.async_copy(tid_hbm.at[rows], tid_v, sem)
    ids_s.wait()
    ids_t.wait()

    for g in range(GROUPS):
        sl = pl.ds(g * 16, 16)
        idx_s_v[sl] = sid_v[sl].astype(jnp.int32)
        idx_t_v[sl] = tid_v[sl].astype(jnp.int32)

    copies = []
    for j in range(N_CHUNKS):
        sl = pl.ds(j * IDX_CHUNK, IDX_CHUNK)
        copies.append(pltpu.async_copy(
            w_s_hbm.at[idx_s_v.at[sl]], emb_s_v.at[sl], sem))
        copies.append(pltpu.async_copy(
            w_t_hbm.at[idx_t_v.at[sl]], emb_t_v.at[sl], sem))
    for c in copies:
        c.wait()

    out_s = pltpu.async_copy(emb_s_v, out_s_hbm.at[rows], sem)
    out_t = pltpu.async_copy(emb_t_v, out_t_hbm.at[rows], sem)
    out_s.wait()
    out_t.wait()


def kernel(x, W_s, W_t):
    emb_s, emb_t = _sc_embed(x[:, F_DENSE], x[:, F_DENSE + 1], W_s, W_t)
    return jnp.concatenate((x[:, :F_DENSE], emb_s, emb_t), axis=1)


# speedup vs baseline: 3.8105x; 1.1788x over previous
import functools

import jax
import jax.numpy as jnp
from jax import lax
from jax.experimental import pallas as pl
from jax.experimental.pallas import tpu as pltpu
from jax.experimental.pallas import tpu_sc as plsc

B = 16384
F_DENSE = 64
VOCAB_S = 100
VOCAB_T = 968
DIM = 32
NC = 2
NS = 16
NW = NC * NS
ROWS_PER_W = B // NW
GROUPS = ROWS_PER_W // 16


@functools.partial(
    pl.kernel,
    out_type=(jax.ShapeDtypeStruct((B, DIM), jnp.float32),
              jax.ShapeDtypeStruct((B, DIM), jnp.float32)),
    mesh=plsc.VectorSubcoreMesh(core_axis_name="c", subcore_axis_name="s"),
    compiler_params=pltpu.CompilerParams(use_tc_tiling_on_sc=False),
    scratch_types=[
        pltpu.VMEM((ROWS_PER_W,), jnp.float32),
        pltpu.VMEM((ROWS_PER_W,), jnp.float32),
        pltpu.VMEM((ROWS_PER_W,), jnp.int32),
        pltpu.VMEM((ROWS_PER_W,), jnp.int32),
        pltpu.VMEM((ROWS_PER_W, DIM), jnp.float32),
        pltpu.VMEM((ROWS_PER_W, DIM), jnp.float32),
        pltpu.VMEM_SHARED((VOCAB_S + VOCAB_T, DIM), jnp.float32),
        pltpu.SemaphoreType.DMA,
    ],
)
def _sc_embed(sid_hbm, tid_hbm, w_s_hbm, w_t_hbm, out_s_hbm, out_t_hbm,
              sid_v, tid_v, idx_s_v, idx_t_v, emb_s_v, emb_t_v, tbl_sh, sem):
    sub = lax.axis_index("s")
    wid = sub * NC + lax.axis_index("c")
    base = wid * ROWS_PER_W
    rows = pl.ds(base, ROWS_PER_W)

    ids_s = pltpu.async_copy(sid_hbm.at[rows], sid_v, sem)
    ids_t = pltpu.async_copy(tid_hbm.at[rows], tid_v, sem)

    @pl.when(sub == 0)
    def _stage_tables():
        pltpu.sync_copy(w_s_hbm, tbl_sh.at[pl.ds(0, VOCAB_S)])
        pltpu.sync_copy(w_t_hbm, tbl_sh.at[pl.ds(VOCAB_S, VOCAB_T)])

    ids_s.wait()
    ids_t.wait()

    for g in range(GROUPS):
        sl = pl.ds(g * 16, 16)
        idx_s_v[sl] = sid_v[sl].astype(jnp.int32)
        idx_t_v[sl] = tid_v[sl].astype(jnp.int32) + VOCAB_S

    plsc.subcore_barrier()

    g_s = pltpu.async_copy(tbl_sh.at[idx_s_v], emb_s_v, sem)
    g_t = pltpu.async_copy(tbl_sh.at[idx_t_v], emb_t_v, sem)
    g_s.wait()
    g_t.wait()

    out_s = pltpu.async_copy(emb_s_v, out_s_hbm.at[rows], sem)
    out_t = pltpu.async_copy(emb_t_v, out_t_hbm.at[rows], sem)
    out_s.wait()
    out_t.wait()


def kernel(x, W_s, W_t):
    emb_s, emb_t = _sc_embed(x[:, F_DENSE], x[:, F_DENSE + 1], W_s, W_t)
    return jnp.concatenate((x[:, :F_DENSE], emb_s, emb_t), axis=1)
